# baseline (speedup 1.0000x reference)
import jax
import jax.numpy as jnp
from jax.experimental import pallas as pl
from jax.experimental.pallas import tpu as pltpu
from jax.experimental.pallas import tpu_sc as plsc

N_E = 8192
C = 32
N_CB = 2
BETA = 0.25
EPS = 1e-12
ST = 1024
N_ST = N_E // ST
GW = 128
GP = 128


def _rownorm(x):
    return x / jnp.maximum(jnp.sqrt(jnp.sum(x * x, axis=1, keepdims=True)), EPS)


def _dist_kernel(z_ref, w_ref, idx_ref, wn0_ref, wn_ref, s0_ref, s1_ref):
    p = pl.program_id(0)

    @pl.when(p == 0)
    def _():
        wn_ref[...] = _rownorm(w_ref[...])
        wn0_ref[...] = jnp.zeros_like(wn0_ref)
        wn0_ref[:, 0:C] = wn_ref[0:N_E, :]

    z = z_ref[0]
    n = z.shape[0]
    zn = (_rownorm(z[:, :C]), _rownorm(z[:, C:]))
    dnums = (((1,), (1,)), ((), ()))

    sbufs = (s0_ref, s1_ref)
    best = {}
    for f in range(N_CB):
        best[f] = (jnp.full((1, n), -jnp.inf, jnp.float32),
                   jnp.zeros((1, n), jnp.int32))
    iota = jax.lax.broadcasted_iota(jnp.int32, (ST, n), 0)
    k = 0
    for ch in range(N_E // ST):
        for f in range(N_CB):
            sb = sbufs[k % 2]
            k += 1
            lo = f * N_E + ch * ST
            sb[...] = jax.lax.dot_general(wn_ref[lo:lo + ST, :], zn[f], dnums,
                                          preferred_element_type=jnp.float32)
            s = sb[...]
            m = jnp.max(s, axis=0, keepdims=True)
            ti = jnp.min(jnp.where(s == m, iota, N_CB * N_E),
                         axis=0, keepdims=True) + ch * ST
            best_m, best_i = best[f]
            upd = m > best_m
            best[f] = (jnp.where(upd, m, best_m), jnp.where(upd, ti, best_i))
    for f in range(N_CB):
        idx_ref[0, f, :] = best[f][1][0]


def _epi_kernel(z_ref, zq_ref, out_ref, loss_ref):
    p = pl.program_id(0)
    z = z_ref[0]
    zn0 = _rownorm(z[:, :C])
    zn1 = _rownorm(z[:, C:])
    zq0 = zq_ref[0, 0][:, :C]
    zq1 = zq_ref[0, 1][:, :C]
    out_ref[0] = jnp.concatenate([zn0 + (zq0 - zn0), zn1 + (zq1 - zn1)], axis=1)
    part = (jnp.sum((zq0 - zn0) ** 2, axis=(0, 1), keepdims=True)
            + jnp.sum((zq1 - zn1) ** 2, axis=(0, 1), keepdims=True))

    @pl.when(p == 0)
    def _():
        loss_ref[...] = part

    @pl.when(p > 0)
    def _():
        loss_ref[...] += part


def _sc_gather(wn0, idx_flat):
    num = idx_flat.shape[1]
    mesh = plsc.VectorSubcoreMesh(core_axis_name="c", subcore_axis_name="s")

    @pl.kernel(out_type=jax.ShapeDtypeStruct((num, GP), jnp.float32), mesh=mesh)
    def gk(wn0_hbm, i_hbm, o_hbm):
        def gbody(i_vmem, o_vmem):
            pltpu.sync_copy(wn0_hbm.at[i_vmem.at[0]], o_vmem)

        pltpu.emit_pipeline(
            gbody,
            grid=(num // GW,),
            in_specs=[pl.BlockSpec((1, GW), index_map=lambda i: (0, i))],
            out_specs=[pl.BlockSpec((GW, GP), index_map=lambda i: (i, 0))],
            core_axis_name=("c", "s"),
            dimension_semantics=(pltpu.PARALLEL,),
        )(i_hbm, o_hbm)

    return gk(wn0, idx_flat)


def kernel(z, W):
    bs, n, e_dim = z.shape
    idx, wn0 = pl.pallas_call(
        _dist_kernel,
        grid=(bs,),
        in_specs=[
            pl.BlockSpec((1, n, e_dim), lambda p: (p, 0, 0)),
            pl.BlockSpec((N_CB * N_E, C), lambda p: (0, 0)),
        ],
        out_specs=[
            pl.BlockSpec((1, N_CB, n), lambda p: (p, 0, 0)),
            pl.BlockSpec((N_E, GP), lambda p: (0, 0)),
        ],
        out_shape=[
            jax.ShapeDtypeStruct((bs, N_CB, n), jnp.int32),
            jax.ShapeDtypeStruct((N_E, GP), jnp.float32),
        ],
        scratch_shapes=[pltpu.VMEM((N_CB * N_E, C), jnp.float32),
                        pltpu.VMEM((ST, n), jnp.float32),
                        pltpu.VMEM((ST, n), jnp.float32)],
    )(z, W)

    zq = _sc_gather(wn0, idx.reshape(1, bs * N_CB * n))
    zq4 = zq.reshape(bs, N_CB, n, GP)

    out, loss = pl.pallas_call(
        _epi_kernel,
        grid=(bs,),
        in_specs=[
            pl.BlockSpec((1, n, e_dim), lambda p: (p, 0, 0)),
            pl.BlockSpec((1, N_CB, n, GP), lambda p: (p, 0, 0, 0)),
        ],
        out_specs=[
            pl.BlockSpec((1, n, e_dim), lambda p: (p, 0, 0)),
            pl.BlockSpec((1, 1), lambda p: (0, 0)),
        ],
        out_shape=[
            jax.ShapeDtypeStruct((bs, n, e_dim), jnp.float32),
            jax.ShapeDtypeStruct((1, 1), jnp.float32),
        ],
    )(z, zq4)

    loss_val = loss[0, 0] * (BETA + 1.0) / (bs * N_CB * n * C)
    return out, loss_val, idx


# speedup vs baseline: 1.0001x; 1.0001x over previous
import jax
import jax.numpy as jnp
from jax.experimental import pallas as pl
from jax.experimental.pallas import tpu as pltpu
from jax.experimental.pallas import tpu_sc as plsc

N_E = 8192
C = 32
N_CB = 2
BETA = 0.25
EPS = 1e-12
ST = 1024
N_ST = N_E // ST
GW = 128
GP = 128


def _rownorm(x):
    return x / jnp.maximum(jnp.sqrt(jnp.sum(x * x, axis=1, keepdims=True)), EPS)


def _dist_kernel(z_ref, w_ref, idx_ref, wn0_ref, wn_ref, s0_ref, s1_ref):
    p = pl.program_id(0)

    @pl.when(p == 0)
    def _():
        wn_ref[...] = _rownorm(w_ref[...])
        wn0_ref[...] = jnp.zeros_like(wn0_ref)
        wn0_ref[:, 0:C] = wn_ref[0:N_E, :]

    z = z_ref[0]
    n = z.shape[0]
    zn = (_rownorm(z[:, :C]).T, _rownorm(z[:, C:]).T)
    dnums = (((1,), (0,)), ((), ()))

    sbufs = (s0_ref, s1_ref)
    best = {}
    for f in range(N_CB):
        best[f] = (jnp.full((1, n), -jnp.inf, jnp.float32),
                   jnp.zeros((1, n), jnp.int32))
    iota = jax.lax.broadcasted_iota(jnp.int32, (ST, n), 0)
    k = 0
    for ch in range(N_E // ST):
        for f in range(N_CB):
            sb = sbufs[k % 2]
            k += 1
            lo = f * N_E + ch * ST
            sb[...] = jax.lax.dot_general(wn_ref[lo:lo + ST, :], zn[f], dnums,
                                          preferred_element_type=jnp.float32)
            s = sb[...]
            m = jnp.max(s, axis=0, keepdims=True)
            ti = jnp.min(jnp.where(s == m, iota, N_CB * N_E),
                         axis=0, keepdims=True) + ch * ST
            best_m, best_i = best[f]
            upd = m > best_m
            best[f] = (jnp.where(upd, m, best_m), jnp.where(upd, ti, best_i))
    for f in range(N_CB):
        idx_ref[0, f, :] = best[f][1][0]


def _epi_kernel(z_ref, zq_ref, out_ref, loss_ref):
    p = pl.program_id(0)
    z = z_ref[0]
    zn0 = _rownorm(z[:, :C])
    zn1 = _rownorm(z[:, C:])
    zq0 = zq_ref[0, 0][:, :C]
    zq1 = zq_ref[0, 1][:, :C]
    out_ref[0] = jnp.concatenate([zn0 + (zq0 - zn0), zn1 + (zq1 - zn1)], axis=1)
    part = (jnp.sum((zq0 - zn0) ** 2, axis=(0, 1), keepdims=True)
            + jnp.sum((zq1 - zn1) ** 2, axis=(0, 1), keepdims=True))

    @pl.when(p == 0)
    def _():
        loss_ref[...] = part

    @pl.when(p > 0)
    def _():
        loss_ref[...] += part


def _sc_gather(wn0, idx_flat):
    num = idx_flat.shape[1]
    mesh = plsc.VectorSubcoreMesh(core_axis_name="c", subcore_axis_name="s")

    @pl.kernel(out_type=jax.ShapeDtypeStruct((num, GP), jnp.float32), mesh=mesh)
    def gk(wn0_hbm, i_hbm, o_hbm):
        def gbody(i_vmem, o_vmem):
            pltpu.sync_copy(wn0_hbm.at[i_vmem.at[0]], o_vmem)

        pltpu.emit_pipeline(
            gbody,
            grid=(num // GW,),
            in_specs=[pl.BlockSpec((1, GW), index_map=lambda i: (0, i))],
            out_specs=[pl.BlockSpec((GW, GP), index_map=lambda i: (i, 0))],
            core_axis_name=("c", "s"),
            dimension_semantics=(pltpu.PARALLEL,),
        )(i_hbm, o_hbm)

    return gk(wn0, idx_flat)


def kernel(z, W):
    bs, n, e_dim = z.shape
    idx, wn0 = pl.pallas_call(
        _dist_kernel,
        grid=(bs,),
        in_specs=[
            pl.BlockSpec((1, n, e_dim), lambda p: (p, 0, 0)),
            pl.BlockSpec((N_CB * N_E, C), lambda p: (0, 0)),
        ],
        out_specs=[
            pl.BlockSpec((1, N_CB, n), lambda p: (p, 0, 0)),
            pl.BlockSpec((N_E, GP), lambda p: (0, 0)),
        ],
        out_shape=[
            jax.ShapeDtypeStruct((bs, N_CB, n), jnp.int32),
            jax.ShapeDtypeStruct((N_E, GP), jnp.float32),
        ],
        scratch_shapes=[pltpu.VMEM((N_CB * N_E, C), jnp.float32),
                        pltpu.VMEM((ST, n), jnp.float32),
                        pltpu.VMEM((ST, n), jnp.float32)],
    )(z, W)

    zq = _sc_gather(wn0, idx.reshape(1, bs * N_CB * n))
    zq4 = zq.reshape(bs, N_CB, n, GP)

    out, loss = pl.pallas_call(
        _epi_kernel,
        grid=(bs,),
        in_specs=[
            pl.BlockSpec((1, n, e_dim), lambda p: (p, 0, 0)),
            pl.BlockSpec((1, N_CB, n, GP), lambda p: (p, 0, 0, 0)),
        ],
        out_specs=[
            pl.BlockSpec((1, n, e_dim), lambda p: (p, 0, 0)),
            pl.BlockSpec((1, 1), lambda p: (0, 0)),
        ],
        out_shape=[
            jax.ShapeDtypeStruct((bs, n, e_dim), jnp.float32),
            jax.ShapeDtypeStruct((1, 1), jnp.float32),
        ],
    )(z, zq4)

    loss_val = loss[0, 0] * (BETA + 1.0) / (bs * N_CB * n * C)
    return out, loss_val, idx


# speedup vs baseline: 1.0612x; 1.0611x over previous
import jax
import jax.numpy as jnp
from jax.experimental import pallas as pl
from jax.experimental.pallas import tpu as pltpu
from jax.experimental.pallas import tpu_sc as plsc

N_E = 8192
C = 32
N_CB = 2
BETA = 0.25
EPS = 1e-12
ST = 1024
N_ST = N_E // ST
GW = 128
GP = 128


def _rownorm(x):
    return x / jnp.maximum(jnp.sqrt(jnp.sum(x * x, axis=1, keepdims=True)), EPS)


def _dist_kernel(z_ref, w_ref, idx_ref, wn0_ref, wn_ref, s0_ref, s1_ref):
    p = pl.program_id(0)

    @pl.when(p == 0)
    def _():
        wn_ref[...] = _rownorm(w_ref[...])
        wn0_ref[...] = jnp.zeros_like(wn0_ref)
        wn0_ref[:, 0:C] = wn_ref[0:N_E, :]

    z = z_ref[0]
    n = z.shape[0]
    zn = (_rownorm(z[:, :C]).T, _rownorm(z[:, C:]).T)
    dnums = (((1,), (0,)), ((), ()))

    sbufs = (s0_ref, s1_ref)
    best = {}
    for f in range(N_CB):
        best[f] = (jnp.full((1, n), -jnp.inf, jnp.float32),
                   jnp.zeros((1, n), jnp.int32))
    iota = jax.lax.broadcasted_iota(jnp.int32, (ST, n), 0).astype(jnp.float32)
    big = jnp.float32(N_CB * N_E)
    k = 0
    for ch in range(N_E // ST):
        for f in range(N_CB):
            sb = sbufs[k % 2]
            k += 1
            lo = f * N_E + ch * ST
            sb[...] = jax.lax.dot_general(wn_ref[lo:lo + ST, :], zn[f], dnums,
                                          preferred_element_type=jnp.float32)
            s = sb[...]
            m = jnp.max(s, axis=0, keepdims=True)
            ti = jnp.min(jnp.where(s == m, iota, big),
                         axis=0, keepdims=True).astype(jnp.int32) + ch * ST
            best_m, best_i = best[f]
            upd = m > best_m
            best[f] = (jnp.where(upd, m, best_m), jnp.where(upd, ti, best_i))
    for f in range(N_CB):
        idx_ref[0, f, :] = best[f][1][0]


def _epi_kernel(z_ref, zq_ref, out_ref, loss_ref):
    p = pl.program_id(0)
    z = z_ref[0]
    zn0 = _rownorm(z[:, :C])
    zn1 = _rownorm(z[:, C:])
    zq0 = zq_ref[0, 0][:, :C]
    zq1 = zq_ref[0, 1][:, :C]
    out_ref[0] = jnp.concatenate([zn0 + (zq0 - zn0), zn1 + (zq1 - zn1)], axis=1)
    part = (jnp.sum((zq0 - zn0) ** 2, axis=(0, 1), keepdims=True)
            + jnp.sum((zq1 - zn1) ** 2, axis=(0, 1), keepdims=True))

    @pl.when(p == 0)
    def _():
        loss_ref[...] = part

    @pl.when(p > 0)
    def _():
        loss_ref[...] += part


def _sc_gather(wn0, idx_flat):
    num = idx_flat.shape[1]
    mesh = plsc.VectorSubcoreMesh(core_axis_name="c", subcore_axis_name="s")

    @pl.kernel(out_type=jax.ShapeDtypeStruct((num, GP), jnp.float32), mesh=mesh)
    def gk(wn0_hbm, i_hbm, o_hbm):
        def gbody(i_vmem, o_vmem):
            pltpu.sync_copy(wn0_hbm.at[i_vmem.at[0]], o_vmem)

        pltpu.emit_pipeline(
            gbody,
            grid=(num // GW,),
            in_specs=[pl.BlockSpec((1, GW), index_map=lambda i: (0, i))],
            out_specs=[pl.BlockSpec((GW, GP), index_map=lambda i: (i, 0))],
            core_axis_name=("c", "s"),
            dimension_semantics=(pltpu.PARALLEL,),
        )(i_hbm, o_hbm)

    return gk(wn0, idx_flat)


def kernel(z, W):
    bs, n, e_dim = z.shape
    idx, wn0 = pl.pallas_call(
        _dist_kernel,
        grid=(bs,),
        in_specs=[
            pl.BlockSpec((1, n, e_dim), lambda p: (p, 0, 0)),
            pl.BlockSpec((N_CB * N_E, C), lambda p: (0, 0)),
        ],
        out_specs=[
            pl.BlockSpec((1, N_CB, n), lambda p: (p, 0, 0)),
            pl.BlockSpec((N_E, GP), lambda p: (0, 0)),
        ],
        out_shape=[
            jax.ShapeDtypeStruct((bs, N_CB, n), jnp.int32),
            jax.ShapeDtypeStruct((N_E, GP), jnp.float32),
        ],
        scratch_shapes=[pltpu.VMEM((N_CB * N_E, C), jnp.float32),
                        pltpu.VMEM((ST, n), jnp.float32),
                        pltpu.VMEM((ST, n), jnp.float32)],
    )(z, W)

    zq = _sc_gather(wn0, idx.reshape(1, bs * N_CB * n))
    zq4 = zq.reshape(bs, N_CB, n, GP)

    out, loss = pl.pallas_call(
        _epi_kernel,
        grid=(bs,),
        in_specs=[
            pl.BlockSpec((1, n, e_dim), lambda p: (p, 0, 0)),
            pl.BlockSpec((1, N_CB, n, GP), lambda p: (p, 0, 0, 0)),
        ],
        out_specs=[
            pl.BlockSpec((1, n, e_dim), lambda p: (p, 0, 0)),
            pl.BlockSpec((1, 1), lambda p: (0, 0)),
        ],
        out_shape=[
            jax.ShapeDtypeStruct((bs, n, e_dim), jnp.float32),
            jax.ShapeDtypeStruct((1, 1), jnp.float32),
        ],
    )(z, zq4)

    loss_val = loss[0, 0] * (BETA + 1.0) / (bs * N_CB * n * C)
    return out, loss_val, idx
